# baseline (speedup 1.0000x reference)
import functools

import jax
import jax.numpy as jnp
from jax import lax
from jax.experimental import pallas as pl
from jax.experimental.pallas import tpu as pltpu
from jax.experimental.pallas import tpu_sc as plsc

_N = 50000
_E = 800000
_D = 128
_H = 64
_HH = 32
_MAX_RECURSION = 10
_TAU = 0.005

_BN = 2000
_NB = _N // _BN

_NC = 2
_NS = 16
_CH = 96
_EPT = _E // _NS
_NCHUNK = _EPT // _CH
_CHT = _EPT - _NCHUNK * _CH
_NPTA = 3128
_NPTB = _N - (_NS - 1) * _NPTA


def _dot_t(a, b):
    return lax.dot_general(a, b, (((1,), (1,)), ((), ())),
                           preferred_element_type=jnp.float32)


_NBUF = 3
_ROUNDS = (_NCHUNK - 1) // _NBUF

_SETW = 10


def _edge_agg(h_flat, row, col, zeros):
    mesh = plsc.VectorSubcoreMesh(core_axis_name="c", subcore_axis_name="s",
                                  num_cores=_NC, num_subcores=_NS)
    scratch = []
    for _ in range(_NBUF):
        scratch += [
            pltpu.VMEM((_CH,), jnp.int32),
            pltpu.VMEM((_CH,), jnp.int32),
            pltpu.VMEM((_CH,), jnp.int32),
            pltpu.VMEM((_CH,), jnp.int32),
            pltpu.VMEM((_CH, _HH), jnp.float32),
            pltpu.VMEM((_CH, _HH), jnp.float32),
            pltpu.VMEM((_CH, _HH), jnp.float32),
            pltpu.SemaphoreType.DMA,
            pltpu.SemaphoreType.DMA,
            pltpu.SemaphoreType.DMA,
        ]
    scratch += [
        pltpu.VMEM((_CHT,), jnp.int32),
        pltpu.VMEM((_CHT,), jnp.int32),
        pltpu.VMEM((_CHT,), jnp.int32),
    ]
    scratch.append(pltpu.VMEM_SHARED((_N, _HH), jnp.float32))

    @functools.partial(
        pl.kernel,
        mesh=mesh,
        compiler_params=pltpu.CompilerParams(use_tc_tiling_on_sc=False),
        out_type=jax.ShapeDtypeStruct((_NC * _N, _HH), jnp.float32),
        scratch_types=scratch,
    )
    def body(h_hbm, row_hbm, col_hbm, z_hbm, out_hbm, *sc):
        sets = [sc[i * _SETW:(i + 1) * _SETW] for i in range(_NBUF)]
        trow_s, trow_g, tcol_g = sc[_NBUF * _SETW:_NBUF * _SETW + 3]
        acc = sc[-1]
        c = lax.axis_index("c")
        s = lax.axis_index("s")
        off = c * _N
        base_n = s * _NPTA

        def issue_idx(st, i):
            base = s * _EPT + i * _CH
            pltpu.async_copy(row_hbm.at[pl.ds(base, _CH)], st[0], st[7])
            pltpu.async_copy(col_hbm.at[pl.ds(base, _CH)], st[2], st[7])

        def wait_idx(st):
            pltpu.make_async_copy(row_hbm.at[pl.ds(0, _CH)], st[0],
                                  st[7]).wait()
            pltpu.make_async_copy(col_hbm.at[pl.ds(0, _CH)], st[2],
                                  st[7]).wait()

        def adjust_and_gather(st):
            for k in range(_CH // 16):
                sl = pl.ds(k * 16, 16)
                st[1][sl] = st[0][sl] + off
                st[2][sl] = st[2][sl] + off
            pltpu.async_copy(h_hbm.at[st[1]], st[4], st[8])
            pltpu.async_copy(h_hbm.at[st[2]], st[5], st[8])

        def wait_gather(st):
            pltpu.make_async_copy(h_hbm.at[st[1]], st[4], st[8]).wait()
            pltpu.make_async_copy(h_hbm.at[st[2]], st[5], st[8]).wait()

        def wait_scatter(st):
            pltpu.make_async_copy(st[6], acc.at[st[3]], st[9]).wait()

        def pin_scatter_idx(st):
            for k in range(_CH // 16):
                sl = pl.ds(k * 16, 16)
                st[3][sl] = st[0][sl]

        def compute_and_scatter(st):
            @plsc.parallel_loop(0, _CH, unroll=8)
            def _(j):
                for k in range(_HH // 16):
                    sl = pl.ds(k * 16, 16)
                    st[6][j, sl] = jnp.abs(st[4][j, sl] - st[5][j, sl])

            pltpu.async_copy(st[6], acc.at[st[3]], st[9], add=True)

        for b in range(_NBUF):
            issue_idx(sets[b], b)

        @pl.when(s < _NS - 1)
        def _():
            pltpu.sync_copy(z_hbm, acc.at[pl.ds(base_n, _NPTA)])

        @pl.when(s == _NS - 1)
        def _():
            pltpu.sync_copy(z_hbm.at[pl.ds(0, _NPTB)],
                            acc.at[pl.ds(base_n, _NPTB)])

        plsc.subcore_barrier()

        for b in range(2):
            wait_idx(sets[b])
            adjust_and_gather(sets[b])

        last = _NCHUNK - 1

        def round_body(g, carry):
            for b in range(_NBUF):
                st = sets[b]
                i = 3 * g + b

                @pl.when(i + 2 <= last)
                def _():
                    st2 = sets[(b + 2) % _NBUF]
                    wait_idx(st2)
                    adjust_and_gather(st2)

                wait_gather(st)

                @pl.when(g > 0)
                def _():
                    wait_scatter(st)

                pin_scatter_idx(st)

                @pl.when(i + 3 <= last)
                def _():
                    issue_idx(st, i + 3)

                compute_and_scatter(st)
            return carry

        lax.fori_loop(0, _ROUNDS, round_body, 0)

        st = sets[last % _NBUF]
        wait_gather(st)
        wait_scatter(st)
        pin_scatter_idx(st)
        compute_and_scatter(st)

        for b in range(_NBUF):
            wait_scatter(sets[b])

        st = sets[0]
        tbase = s * _EPT + _NCHUNK * _CH
        pltpu.sync_copy(row_hbm.at[pl.ds(tbase, _CHT)], trow_s)
        pltpu.sync_copy(col_hbm.at[pl.ds(tbase, _CHT)], tcol_g)
        for k in range(_CHT // 16):
            sl = pl.ds(k * 16, 16)
            trow_g[sl] = trow_s[sl] + off
            tcol_g[sl] = tcol_g[sl] + off
        cp_r = pltpu.async_copy(h_hbm.at[trow_g], st[4].at[pl.ds(0, _CHT)],
                                st[8])
        cp_c = pltpu.async_copy(h_hbm.at[tcol_g], st[5].at[pl.ds(0, _CHT)],
                                st[8])
        cp_r.wait()
        cp_c.wait()

        @plsc.parallel_loop(0, _CHT, unroll=8)
        def _(j):
            for k in range(_HH // 16):
                sl = pl.ds(k * 16, 16)
                st[4][j, sl] = jnp.abs(st[4][j, sl] - st[5][j, sl])

        for k in range(_CHT // 16):
            sl = pl.ds(k * 16, 16)
            trow_g[sl] = trow_s[sl]
        pltpu.sync_copy(st[4].at[pl.ds(0, _CHT)], acc.at[trow_g], add=True)

        plsc.subcore_barrier()

        @pl.when(s < _NS - 1)
        def _():
            pltpu.sync_copy(acc.at[pl.ds(base_n, _NPTA)],
                            out_hbm.at[pl.ds(off + base_n, _NPTA)])

        @pl.when(s == _NS - 1)
        def _():
            pltpu.sync_copy(acc.at[pl.ds(base_n, _NPTB)],
                            out_hbm.at[pl.ds(off + base_n, _NPTB)])

    return body(h_flat, row, col, zeros)


def _in_proj(x, w_in, b_in):
    def kern(x_ref, w_ref, b_ref, hs_ref):
        h = jnp.maximum(_dot_t(x_ref[...], w_ref[...]) + b_ref[...], 0.0)
        hs_ref[0] = h[:, :_HH]
        hs_ref[1] = h[:, _HH:]

    return pl.pallas_call(
        kern,
        grid=(_NB,),
        in_specs=[
            pl.BlockSpec((_BN, _D), lambda i: (i, 0)),
            pl.BlockSpec((_H, _D), lambda i: (0, 0)),
            pl.BlockSpec((1, _H), lambda i: (0, 0)),
        ],
        out_specs=pl.BlockSpec((_NC, _BN, _HH), lambda i: (0, i, 0)),
        out_shape=jax.ShapeDtypeStruct((_NC, _N, _HH), jnp.float32),
    )(x, w_in, b_in)


def _layer(hs, agg, w_l, b_l, w_tau, b_tau, w_ih, w_hh, b_ih, b_hh,
           last, w_out=None, b_out=None):
    def kern(h_ref, a_ref, wl_ref, bl_ref, wt_ref, bt_ref,
             wih_ref, whh_ref, bih_ref, bhh_ref, *rest):
        hb = jnp.concatenate([h_ref[0], h_ref[1]], axis=1)
        ab = jnp.concatenate([a_ref[0], a_ref[1]], axis=1)
        cat = jnp.concatenate([hb, ab], axis=1)
        h2 = jnp.maximum(_dot_t(cat, wl_ref[...]) + bl_ref[...], 0.0)
        tl = _dot_t(h2, wt_ref[...])[:, :1] + bt_ref[...]
        tau = jax.nn.softplus(tl)
        n_upd = jnp.minimum(jnp.floor(1.0 / tau).astype(jnp.int32),
                            _MAX_RECURSION)
        mask = (n_upd > 0) & (tau < _TAU)

        gi = _dot_t(ab, wih_ref[...]) + bih_ref[...]
        gh = _dot_t(h2, whh_ref[...]) + bhh_ref[...]
        r = jax.nn.sigmoid(gi[:, :_H] + gh[:, :_H])
        z = jax.nn.sigmoid(gi[:, _H:2 * _H] + gh[:, _H:2 * _H])
        n = jnp.tanh(gi[:, 2 * _H:] + r * gh[:, 2 * _H:])
        h_upd = (1.0 - z) * n + z * h2
        h_out = jnp.where(mask, h_upd, h2)

        if last:
            wo_ref, bo_ref, out_ref, tau_ref = rest
            out_ref[...] = _dot_t(h_out, wo_ref[...]) + bo_ref[...]
            tau_ref[...] = tau
        else:
            (hs_ref,) = rest
            hs_ref[0] = h_out[:, :_HH]
            hs_ref[1] = h_out[:, _HH:]

    full = lambda shape: pl.BlockSpec(shape, lambda i: tuple(0 for _ in shape))
    in_specs = [
        pl.BlockSpec((_NC, _BN, _HH), lambda i: (0, i, 0)),
        pl.BlockSpec((_NC, _BN, _HH), lambda i: (0, i, 0)),
        full((_H, _D)),
        full((1, _H)),
        full((8, _H)),
        full((1, 1)),
        full((3 * _H, _H)),
        full((3 * _H, _H)),
        full((1, 3 * _H)),
        full((1, 3 * _H)),
    ]
    args = [hs, agg, w_l, b_l, w_tau, b_tau, w_ih, w_hh, b_ih, b_hh]
    if last:
        in_specs += [full((_H, _H)), full((1, _H))]
        args += [w_out, b_out]
        out_specs = [
            pl.BlockSpec((_BN, _H), lambda i: (i, 0)),
            pl.BlockSpec((_BN, 1), lambda i: (i, 0)),
        ]
        out_shape = [
            jax.ShapeDtypeStruct((_N, _H), jnp.float32),
            jax.ShapeDtypeStruct((_N, 1), jnp.float32),
        ]
    else:
        out_specs = pl.BlockSpec((_NC, _BN, _HH), lambda i: (0, i, 0))
        out_shape = jax.ShapeDtypeStruct((_NC, _N, _HH), jnp.float32)

    return pl.pallas_call(
        kern,
        grid=(_NB,),
        in_specs=in_specs,
        out_specs=out_specs,
        out_shape=out_shape,
    )(*args)


def kernel(x, edge_index, W_in, b_in, W_d0, b_d0, W_d1, b_d1, W_tau, b_tau,
           W_ih, W_hh, b_ih, b_hh, W_out, b_out):
    row = edge_index[0]
    col = edge_index[1]
    zeros = jnp.zeros((_NPTA, _HH), jnp.float32)

    hsplit = _in_proj(x, W_in, b_in.reshape(1, _H))

    b_tau2 = b_tau.reshape(1, 1)
    bih = b_ih.reshape(1, 3 * _H)
    bhh = b_hh.reshape(1, 3 * _H)
    w_tau_p = jnp.zeros((8, _H), jnp.float32).at[0].set(W_tau[0])

    agg = _edge_agg(hsplit.reshape(_NC * _N, _HH), row, col, zeros)
    agg = agg.reshape(_NC, _N, _HH)
    hsplit = _layer(hsplit, agg, W_d0, b_d0.reshape(1, _H), w_tau_p, b_tau2,
                    W_ih, W_hh, bih, bhh, last=False)

    agg = _edge_agg(hsplit.reshape(_NC * _N, _HH), row, col, zeros)
    agg = agg.reshape(_NC, _N, _HH)
    out, tau = _layer(hsplit, agg, W_d1, b_d1.reshape(1, _H), w_tau_p, b_tau2,
                      W_ih, W_hh, bih, bhh, last=True,
                      w_out=W_out, b_out=b_out.reshape(1, _H))
    return out, tau.reshape(_N)


# speedup vs baseline: 1.0099x; 1.0099x over previous
import functools

import jax
import jax.numpy as jnp
from jax import lax
from jax.experimental import pallas as pl
from jax.experimental.pallas import tpu as pltpu
from jax.experimental.pallas import tpu_sc as plsc

_N = 50000
_E = 800000
_D = 128
_H = 64
_HH = 32
_MAX_RECURSION = 10
_TAU = 0.005

_BN = 2000
_NB = _N // _BN

_NC = 2
_NS = 16
_CH = 128
_EPT = _E // _NS
_NCHUNK = _EPT // _CH
_CHT = _EPT - _NCHUNK * _CH
_NPTA = 3128
_NPTB = _N - (_NS - 1) * _NPTA


def _dot_t(a, b):
    return lax.dot_general(a, b, (((1,), (1,)), ((), ())),
                           preferred_element_type=jnp.float32)


_NBUF = 3
_ROUNDS = _NCHUNK // _NBUF

_SETW = 8


def _edge_agg(h_flat, row, col, zeros):
    mesh = plsc.VectorSubcoreMesh(core_axis_name="c", subcore_axis_name="s",
                                  num_cores=_NC, num_subcores=_NS)
    scratch = []
    for _ in range(_NBUF):
        scratch += [
            pltpu.VMEM((_CH,), jnp.int32),
            pltpu.VMEM((_CH,), jnp.int32),
            pltpu.VMEM((_CH,), jnp.int32),
            pltpu.VMEM((_CH, _HH), jnp.float32),
            pltpu.VMEM((_CH, _HH), jnp.float32),
            pltpu.SemaphoreType.DMA,
            pltpu.SemaphoreType.DMA,
            pltpu.SemaphoreType.DMA,
        ]
    scratch += [
        pltpu.VMEM((_CHT,), jnp.int32),
        pltpu.VMEM((_CHT,), jnp.int32),
        pltpu.VMEM((_CHT,), jnp.int32),
    ]
    scratch.append(pltpu.VMEM_SHARED((_N, _HH), jnp.float32))

    @functools.partial(
        pl.kernel,
        mesh=mesh,
        compiler_params=pltpu.CompilerParams(use_tc_tiling_on_sc=False),
        out_type=jax.ShapeDtypeStruct((_NC * _N, _HH), jnp.float32),
        scratch_types=scratch,
    )
    def body(h_hbm, row_hbm, col_hbm, z_hbm, out_hbm, *sc):
        sets = [sc[i * _SETW:(i + 1) * _SETW] for i in range(_NBUF)]
        trow_s, trow_g, tcol_g = sc[_NBUF * _SETW:_NBUF * _SETW + 3]
        acc = sc[-1]
        c = lax.axis_index("c")
        s = lax.axis_index("s")
        off = c * _N
        base_n = s * _NPTA

        def issue_idx(st, i):
            base = s * _EPT + i * _CH
            pltpu.async_copy(row_hbm.at[pl.ds(base, _CH)], st[0], st[5])
            pltpu.async_copy(col_hbm.at[pl.ds(base, _CH)], st[2], st[5])

        for b in range(_NBUF):
            issue_idx(sets[b], b)

        @pl.when(s < _NS - 1)
        def _():
            pltpu.sync_copy(z_hbm, acc.at[pl.ds(base_n, _NPTA)])

        @pl.when(s == _NS - 1)
        def _():
            pltpu.sync_copy(z_hbm.at[pl.ds(0, _NPTB)],
                            acc.at[pl.ds(base_n, _NPTB)])

        plsc.subcore_barrier()

        def round_body(g, carry):
            for b in range(_NBUF):
                st = sets[b]
                pltpu.make_async_copy(row_hbm.at[pl.ds(0, _CH)], st[0],
                                      st[5]).wait()
                pltpu.make_async_copy(col_hbm.at[pl.ds(0, _CH)], st[2],
                                      st[5]).wait()

                @pl.when(g > 0)
                def _():
                    pltpu.make_async_copy(st[3], acc.at[st[1]], st[7]).wait()

                for k in range(_CH // 16):
                    sl = pl.ds(k * 16, 16)
                    st[1][sl] = st[0][sl] + off
                    st[2][sl] = st[2][sl] + off
                pltpu.async_copy(h_hbm.at[st[1]], st[3], st[6])
                pltpu.async_copy(h_hbm.at[st[2]], st[4], st[6])

            for b in range(_NBUF):
                st = sets[b]
                pltpu.make_async_copy(h_hbm.at[st[1]], st[3], st[6]).wait()
                pltpu.make_async_copy(h_hbm.at[st[2]], st[4], st[6]).wait()

                for k in range(_CH // 16):
                    sl = pl.ds(k * 16, 16)
                    st[1][sl] = st[0][sl]

                @plsc.parallel_loop(0, _CH, unroll=4)
                def _(j):
                    for k in range(_HH // 16):
                        sl = pl.ds(k * 16, 16)
                        st[3][j, sl] = jnp.abs(st[3][j, sl] - st[4][j, sl])

                pltpu.async_copy(st[3], acc.at[st[1]], st[7], add=True)

                @pl.when(g < _ROUNDS - 1)
                def _():
                    issue_idx(st, (g + 1) * _NBUF + b)
            return carry

        lax.fori_loop(0, _ROUNDS, round_body, 0)

        for b in range(_NBUF):
            st = sets[b]
            pltpu.make_async_copy(st[3], acc.at[st[1]], st[7]).wait()

        st = sets[0]
        tbase = s * _EPT + _NCHUNK * _CH
        pltpu.sync_copy(row_hbm.at[pl.ds(tbase, _CHT)], trow_s)
        pltpu.sync_copy(col_hbm.at[pl.ds(tbase, _CHT)], tcol_g)
        for k in range(_CHT // 16):
            sl = pl.ds(k * 16, 16)
            trow_g[sl] = trow_s[sl] + off
            tcol_g[sl] = tcol_g[sl] + off
        cp_r = pltpu.async_copy(h_hbm.at[trow_g], st[3].at[pl.ds(0, _CHT)],
                                st[6])
        cp_c = pltpu.async_copy(h_hbm.at[tcol_g], st[4].at[pl.ds(0, _CHT)],
                                st[6])
        cp_r.wait()
        cp_c.wait()

        @plsc.parallel_loop(0, _CHT, unroll=4)
        def _(j):
            for k in range(_HH // 16):
                sl = pl.ds(k * 16, 16)
                st[3][j, sl] = jnp.abs(st[3][j, sl] - st[4][j, sl])

        for k in range(_CHT // 16):
            sl = pl.ds(k * 16, 16)
            trow_g[sl] = trow_s[sl]
        pltpu.sync_copy(st[3].at[pl.ds(0, _CHT)], acc.at[trow_g], add=True)

        plsc.subcore_barrier()

        @pl.when(s < _NS - 1)
        def _():
            pltpu.sync_copy(acc.at[pl.ds(base_n, _NPTA)],
                            out_hbm.at[pl.ds(off + base_n, _NPTA)])

        @pl.when(s == _NS - 1)
        def _():
            pltpu.sync_copy(acc.at[pl.ds(base_n, _NPTB)],
                            out_hbm.at[pl.ds(off + base_n, _NPTB)])

    return body(h_flat, row, col, zeros)


def _in_proj(x, w_in, b_in):
    def kern(x_ref, w_ref, b_ref, h_ref, hs_ref):
        h = jnp.maximum(_dot_t(x_ref[...], w_ref[...]) + b_ref[...], 0.0)
        h_ref[...] = h
        hs_ref[0] = h[:, :_HH]
        hs_ref[1] = h[:, _HH:]

    return pl.pallas_call(
        kern,
        grid=(_NB,),
        in_specs=[
            pl.BlockSpec((_BN, _D), lambda i: (i, 0)),
            pl.BlockSpec((_H, _D), lambda i: (0, 0)),
            pl.BlockSpec((1, _H), lambda i: (0, 0)),
        ],
        out_specs=[
            pl.BlockSpec((_BN, _H), lambda i: (i, 0)),
            pl.BlockSpec((_NC, _BN, _HH), lambda i: (0, i, 0)),
        ],
        out_shape=[
            jax.ShapeDtypeStruct((_N, _H), jnp.float32),
            jax.ShapeDtypeStruct((_NC, _N, _HH), jnp.float32),
        ],
    )(x, w_in, b_in)


def _layer(h, agg, w_l, b_l, w_tau, b_tau, w_ih, w_hh, b_ih, b_hh,
           last, w_out=None, b_out=None):
    def kern(h_ref, a_ref, wl_ref, bl_ref, wt_ref, bt_ref,
             wih_ref, whh_ref, bih_ref, bhh_ref, *rest):
        hb = h_ref[...]
        ab = jnp.concatenate([a_ref[0], a_ref[1]], axis=1)
        cat = jnp.concatenate([hb, ab], axis=1)
        h2 = jnp.maximum(_dot_t(cat, wl_ref[...]) + bl_ref[...], 0.0)
        tl = _dot_t(h2, wt_ref[...])[:, :1] + bt_ref[...]
        tau = jax.nn.softplus(tl)
        n_upd = jnp.minimum(jnp.floor(1.0 / tau).astype(jnp.int32),
                            _MAX_RECURSION)
        mask = (n_upd > 0) & (tau < _TAU)

        gi = _dot_t(ab, wih_ref[...]) + bih_ref[...]
        gh = _dot_t(h2, whh_ref[...]) + bhh_ref[...]
        r = jax.nn.sigmoid(gi[:, :_H] + gh[:, :_H])
        z = jax.nn.sigmoid(gi[:, _H:2 * _H] + gh[:, _H:2 * _H])
        n = jnp.tanh(gi[:, 2 * _H:] + r * gh[:, 2 * _H:])
        h_upd = (1.0 - z) * n + z * h2
        h_out = jnp.where(mask, h_upd, h2)

        if last:
            wo_ref, bo_ref, out_ref, tau_ref = rest
            out_ref[...] = _dot_t(h_out, wo_ref[...]) + bo_ref[...]
            tau_ref[...] = tau
        else:
            out_ref, hs_ref = rest
            out_ref[...] = h_out
            hs_ref[0] = h_out[:, :_HH]
            hs_ref[1] = h_out[:, _HH:]

    full = lambda shape: pl.BlockSpec(shape, lambda i: tuple(0 for _ in shape))
    in_specs = [
        pl.BlockSpec((_BN, _H), lambda i: (i, 0)),
        pl.BlockSpec((_NC, _BN, _HH), lambda i: (0, i, 0)),
        full((_H, _D)),
        full((1, _H)),
        full((8, _H)),
        full((1, 1)),
        full((3 * _H, _H)),
        full((3 * _H, _H)),
        full((1, 3 * _H)),
        full((1, 3 * _H)),
    ]
    args = [h, agg, w_l, b_l, w_tau, b_tau, w_ih, w_hh, b_ih, b_hh]
    if last:
        in_specs += [full((_H, _H)), full((1, _H))]
        args += [w_out, b_out]
        out_specs = [
            pl.BlockSpec((_BN, _H), lambda i: (i, 0)),
            pl.BlockSpec((_BN, 1), lambda i: (i, 0)),
        ]
        out_shape = [
            jax.ShapeDtypeStruct((_N, _H), jnp.float32),
            jax.ShapeDtypeStruct((_N, 1), jnp.float32),
        ]
    else:
        out_specs = [
            pl.BlockSpec((_BN, _H), lambda i: (i, 0)),
            pl.BlockSpec((_NC, _BN, _HH), lambda i: (0, i, 0)),
        ]
        out_shape = [
            jax.ShapeDtypeStruct((_N, _H), jnp.float32),
            jax.ShapeDtypeStruct((_NC, _N, _HH), jnp.float32),
        ]

    return pl.pallas_call(
        kern,
        grid=(_NB,),
        in_specs=in_specs,
        out_specs=out_specs,
        out_shape=out_shape,
    )(*args)


def kernel(x, edge_index, W_in, b_in, W_d0, b_d0, W_d1, b_d1, W_tau, b_tau,
           W_ih, W_hh, b_ih, b_hh, W_out, b_out):
    row = edge_index[0]
    col = edge_index[1]
    zeros = jnp.zeros((_NPTA, _HH), jnp.float32)

    h, hsplit = _in_proj(x, W_in, b_in.reshape(1, _H))

    b_tau2 = b_tau.reshape(1, 1)
    bih = b_ih.reshape(1, 3 * _H)
    bhh = b_hh.reshape(1, 3 * _H)
    w_tau_p = jnp.zeros((8, _H), jnp.float32).at[0].set(W_tau[0])

    agg = _edge_agg(hsplit.reshape(_NC * _N, _HH), row, col, zeros)
    agg = agg.reshape(_NC, _N, _HH)
    h, hsplit = _layer(h, agg, W_d0, b_d0.reshape(1, _H), w_tau_p, b_tau2,
                       W_ih, W_hh, bih, bhh, last=False)

    agg = _edge_agg(hsplit.reshape(_NC * _N, _HH), row, col, zeros)
    agg = agg.reshape(_NC, _N, _HH)
    out, tau = _layer(h, agg, W_d1, b_d1.reshape(1, _H), w_tau_p, b_tau2,
                      W_ih, W_hh, bih, bhh, last=True,
                      w_out=W_out, b_out=b_out.reshape(1, _H))
    return out, tau.reshape(_N)
